# initial kernel scaffold (unmeasured)
import jax
import jax.numpy as jnp
from jax import lax
from jax.experimental import pallas as pl
from jax.experimental.pallas import tpu as pltpu


def kernel(x, W):
    t, d = x.shape
    _, v_half = W.shape

    def body(x_ref, w_ref, out_ref, logits_ref, comm_ref, send_sem, recv_sem):
        my_x = lax.axis_index("x")
        my_y = lax.axis_index("y")
        my_z = lax.axis_index("z")
        peer = (my_x, 1 - my_y, my_z)

        barrier = pltpu.get_barrier_semaphore()
        pl.semaphore_signal(
            barrier, inc=1, device_id=peer,
            device_id_type=pl.DeviceIdType.MESH,
        )
        pl.semaphore_wait(barrier, 1)

        xb = x_ref[...].astype(jnp.bfloat16)
        logits_ref[...] = jnp.dot(
            xb, w_ref[...].astype(jnp.bfloat16),
            preferred_element_type=jnp.float32,
        ).astype(jnp.bfloat16)

        rdma = pltpu.make_async_remote_copy(
            src_ref=logits_ref,
            dst_ref=comm_ref,
            send_sem=send_sem,
            recv_sem=recv_sem,
            device_id=peer,
            device_id_type=pl.DeviceIdType.MESH,
        )
        rdma.start()
        rdma.wait()

        m_own = logits_ref[...].astype(jnp.float32).max(axis=-1, keepdims=True)
        m_oth = comm_ref[...].astype(jnp.float32).max(axis=-1, keepdims=True)
        m = jnp.maximum(m_own, m_oth)

        @pl.when(my_y == 0)
        def _():
            out_ref[:, :v_half] = jnp.exp(logits_ref[...].astype(jnp.float32) - m)
            out_ref[:, v_half:] = jnp.exp(comm_ref[...].astype(jnp.float32) - m)

        @pl.when(my_y == 1)
        def _():
            out_ref[:, :v_half] = jnp.exp(comm_ref[...].astype(jnp.float32) - m)
            out_ref[:, v_half:] = jnp.exp(logits_ref[...].astype(jnp.float32) - m)

        s = jnp.sum(out_ref[...], axis=-1, keepdims=True)
        out_ref[...] = out_ref[...] / s

    return pl.pallas_call(
        body,
        out_shape=jax.ShapeDtypeStruct((t, 2 * v_half), jnp.float32),
        in_specs=[
            pl.BlockSpec(memory_space=pltpu.VMEM),
            pl.BlockSpec(memory_space=pltpu.VMEM),
        ],
        out_specs=pl.BlockSpec(memory_space=pltpu.VMEM),
        scratch_shapes=[
            pltpu.VMEM((t, v_half), jnp.bfloat16),
            pltpu.VMEM((t, v_half), jnp.bfloat16),
            pltpu.SemaphoreType.DMA,
            pltpu.SemaphoreType.DMA,
        ],
        compiler_params=pltpu.CompilerParams(collective_id=0),
    )(x, W)


# baseline (device time: 167911 ns/iter reference)
import jax
import jax.numpy as jnp
from jax import lax
from jax.experimental import pallas as pl
from jax.experimental.pallas import tpu as pltpu


def kernel(x, W):
    t, d = x.shape
    _, v_half = W.shape

    CH = 1024

    def body(x_ref, w_hbm, out_ref, logits_ref, comm_ref, wbuf, copy_sem,
             send_sem, recv_sem):
        my_x = lax.axis_index("x")
        my_y = lax.axis_index("y")
        my_z = lax.axis_index("z")
        peer = (my_x, 1 - my_y, my_z)

        barrier = pltpu.get_barrier_semaphore()
        pl.semaphore_signal(
            barrier, inc=1, device_id=peer,
            device_id_type=pl.DeviceIdType.MESH,
        )
        pl.semaphore_wait(barrier, 1)

        xb = x_ref[...].astype(jnp.bfloat16)
        for k in range(v_half // CH):
            cp = pltpu.make_async_copy(
                w_hbm.at[:, pl.ds(k * CH, CH)], wbuf, copy_sem
            )
            cp.start()
            cp.wait()
            logits_ref[:, k * CH:(k + 1) * CH] = jnp.dot(
                xb, wbuf[...].astype(jnp.bfloat16),
                preferred_element_type=jnp.float32,
            ).astype(jnp.bfloat16)

        rdma = pltpu.make_async_remote_copy(
            src_ref=logits_ref,
            dst_ref=comm_ref,
            send_sem=send_sem,
            recv_sem=recv_sem,
            device_id=peer,
            device_id_type=pl.DeviceIdType.MESH,
        )
        rdma.start()
        rdma.wait()

        m_own = logits_ref[...].astype(jnp.float32).max(axis=-1, keepdims=True)
        m_oth = comm_ref[...].astype(jnp.float32).max(axis=-1, keepdims=True)
        m = jnp.maximum(m_own, m_oth)

        @pl.when(my_y == 0)
        def _():
            out_ref[:, :v_half] = jnp.exp(logits_ref[...].astype(jnp.float32) - m)
            out_ref[:, v_half:] = jnp.exp(comm_ref[...].astype(jnp.float32) - m)

        @pl.when(my_y == 1)
        def _():
            out_ref[:, :v_half] = jnp.exp(comm_ref[...].astype(jnp.float32) - m)
            out_ref[:, v_half:] = jnp.exp(logits_ref[...].astype(jnp.float32) - m)

        s = jnp.sum(out_ref[...], axis=-1, keepdims=True)
        out_ref[...] = out_ref[...] / s

    return pl.pallas_call(
        body,
        out_shape=jax.ShapeDtypeStruct((t, 2 * v_half), jnp.float32),
        in_specs=[
            pl.BlockSpec(memory_space=pltpu.VMEM),
            pl.BlockSpec(memory_space=pl.ANY),
        ],
        out_specs=pl.BlockSpec(memory_space=pltpu.VMEM),
        scratch_shapes=[
            pltpu.VMEM((t, v_half), jnp.bfloat16),
            pltpu.VMEM((t, v_half), jnp.bfloat16),
            pltpu.VMEM((d, CH), jnp.float32),
            pltpu.SemaphoreType.DMA,
            pltpu.SemaphoreType.DMA,
            pltpu.SemaphoreType.DMA,
        ],
        compiler_params=pltpu.CompilerParams(
            collective_id=0,
            vmem_limit_bytes=62 * 1024 * 1024,
        ),
    )(x, W)


# device time: 136153 ns/iter; 1.2333x vs baseline; 1.2333x over previous
import functools

import jax
import jax.numpy as jnp
from jax import lax
from jax.experimental import pallas as pl
from jax.experimental.pallas import tpu as pltpu


def kernel(x, W):
    t, d = x.shape
    _, v_half = W.shape
    CH = 1024
    NCH = v_half // CH
    f32 = jnp.float32

    def body(x_ref, w_hbm, out_ref, logits_ref, comm_ref, wbuf, wsems,
             send_sems, recv_sems):
        my_x = lax.axis_index("x")
        my_y = lax.axis_index("y")
        my_z = lax.axis_index("z")
        peer = (my_x, 1 - my_y, my_z)

        barrier = pltpu.get_barrier_semaphore()
        pl.semaphore_signal(
            barrier, inc=1, device_id=peer,
            device_id_type=pl.DeviceIdType.MESH,
        )
        pl.semaphore_wait(barrier, 1)

        xb = x_ref[...].astype(jnp.bfloat16)

        def w_copy(k):
            return pltpu.make_async_copy(
                w_hbm.at[:, pl.ds(k * CH, CH)], wbuf.at[k % 2], wsems.at[k % 2]
            )

        w_copy(0).start()
        rdmas = []
        m_parts = []
        for k in range(NCH):
            if k + 1 < NCH:
                w_copy(k + 1).start()
            w_copy(k).wait()
            acc = jnp.dot(
                xb, wbuf[k % 2].astype(jnp.bfloat16),
                preferred_element_type=f32,
            )
            logits_ref[k] = acc.astype(jnp.bfloat16)
            m_parts.append(acc.max(axis=-1, keepdims=True))
            rdma = pltpu.make_async_remote_copy(
                src_ref=logits_ref.at[k],
                dst_ref=comm_ref.at[k],
                send_sem=send_sems.at[k],
                recv_sem=recv_sems.at[k],
                device_id=peer,
                device_id_type=pl.DeviceIdType.MESH,
            )
            rdma.start()
            rdmas.append(rdma)

        m_own = functools.reduce(jnp.maximum, m_parts)

        def tail(own_base, oth_base):
            s_own = None
            for k in range(NCH):
                e = jnp.exp(logits_ref[k].astype(f32) - m_own)
                out_ref[:, own_base + k * CH:own_base + (k + 1) * CH] = (
                    e.astype(out_ref.dtype))
                es = jnp.sum(e, axis=-1, keepdims=True)
                s_own = es if s_own is None else s_own + es
            m_oth, s_oth = [], []
            for j in range(NCH):
                rdmas[j].wait_recv()
                c = comm_ref[j].astype(f32)
                m_j = c.max(axis=-1, keepdims=True)
                e = jnp.exp(c - m_j)
                out_ref[:, oth_base + j * CH:oth_base + (j + 1) * CH] = (
                    e.astype(out_ref.dtype))
                m_oth.append(m_j)
                s_oth.append(jnp.sum(e, axis=-1, keepdims=True))
            m = functools.reduce(jnp.maximum, m_oth + [m_own])
            s = s_own * jnp.exp(m_own - m)
            for j in range(NCH):
                s = s + s_oth[j] * jnp.exp(m_oth[j] - m)
            r = 1.0 / s
            a_own = jnp.exp(m_own - m) * r
            sl = slice(own_base, own_base + v_half)
            out_ref[:, sl] = (out_ref[:, sl] * a_own).astype(out_ref.dtype)
            for j in range(NCH):
                a_j = jnp.exp(m_oth[j] - m) * r
                sl = slice(oth_base + j * CH, oth_base + (j + 1) * CH)
                out_ref[:, sl] = (out_ref[:, sl] * a_j).astype(out_ref.dtype)

        @pl.when(my_y == 0)
        def _():
            tail(0, v_half)

        @pl.when(my_y == 1)
        def _():
            tail(v_half, 0)

        for k in range(NCH):
            rdmas[k].wait_send()

    return pl.pallas_call(
        body,
        out_shape=jax.ShapeDtypeStruct((t, 2 * v_half), jnp.bfloat16),
        in_specs=[
            pl.BlockSpec(memory_space=pltpu.VMEM),
            pl.BlockSpec(memory_space=pl.ANY),
        ],
        out_specs=pl.BlockSpec(memory_space=pltpu.VMEM),
        scratch_shapes=[
            pltpu.VMEM((NCH, t, CH), jnp.bfloat16),
            pltpu.VMEM((NCH, t, CH), jnp.bfloat16),
            pltpu.VMEM((2, d, CH), jnp.float32),
            pltpu.SemaphoreType.DMA((2,)),
            pltpu.SemaphoreType.DMA((NCH,)),
            pltpu.SemaphoreType.DMA((NCH,)),
        ],
        compiler_params=pltpu.CompilerParams(
            collective_id=0,
            vmem_limit_bytes=67_000_000,
        ),
    )(x, W)


# device time: 118641 ns/iter; 1.4153x vs baseline; 1.1476x over previous
import functools
import os

import jax
import jax.numpy as jnp
from jax import lax
from jax.experimental import pallas as pl
from jax.experimental.pallas import tpu as pltpu


def kernel(x, W):
    t, d = x.shape
    _, v_half = W.shape
    CH = 1024
    NCH = v_half // CH
    f32 = jnp.float32

    def body(x_ref, w_hbm, out_ref, logits_ref, comm_ref, wbuf, wsems,
             send_sems, recv_sems):
        my_x = lax.axis_index("x")
        my_y = lax.axis_index("y")
        my_z = lax.axis_index("z")
        peer = (my_x, 1 - my_y, my_z)

        barrier = pltpu.get_barrier_semaphore()
        pl.semaphore_signal(
            barrier, inc=1, device_id=peer,
            device_id_type=pl.DeviceIdType.MESH,
        )
        pl.semaphore_wait(barrier, 1)

        xb = x_ref[...].astype(jnp.bfloat16)

        def w_copy(k):
            return pltpu.make_async_copy(
                w_hbm.at[:, pl.ds(k * CH, CH)], wbuf.at[k % 2], wsems.at[k % 2]
            )

        w_copy(0).start()
        rdmas = []
        m_parts = []
        for k in range(NCH):
            if k + 1 < NCH:
                w_copy(k + 1).start()
            w_copy(k).wait()
            acc = jnp.dot(
                xb, wbuf[k % 2].astype(jnp.bfloat16),
                preferred_element_type=f32,
            )
            logits_ref[k] = acc.astype(jnp.bfloat16)
            m_parts.append(acc.max(axis=-1, keepdims=True))
            rdma = pltpu.make_async_remote_copy(
                src_ref=logits_ref.at[k],
                dst_ref=comm_ref.at[k],
                send_sem=send_sems.at[k],
                recv_sem=recv_sems.at[k],
                device_id=peer,
                device_id_type=pl.DeviceIdType.MESH,
            )
            rdma.start()
            rdmas.append(rdma)

        m_own = functools.reduce(jnp.maximum, m_parts)

        def tail(own_base, oth_base):
            s_own = None
            for k in range(NCH):
                e = jnp.exp(logits_ref[k].astype(f32) - m_own)
                out_ref[:, own_base + k * CH:own_base + (k + 1) * CH] = (
                    e.astype(out_ref.dtype))
                es = jnp.sum(e, axis=-1, keepdims=True)
                s_own = es if s_own is None else s_own + es
            m_oth, s_oth = [], []
            for j in range(NCH):
                rdmas[j].wait_recv()
                c = comm_ref[j].astype(f32)
                m_j = c.max(axis=-1, keepdims=True)
                e = jnp.exp(c - m_j)
                out_ref[:, oth_base + j * CH:oth_base + (j + 1) * CH] = (
                    e.astype(out_ref.dtype))
                m_oth.append(m_j)
                s_oth.append(jnp.sum(e, axis=-1, keepdims=True))
            m = functools.reduce(jnp.maximum, m_oth + [m_own])
            s = s_own * jnp.exp(m_own - m)
            for j in range(NCH):
                s = s + s_oth[j] * jnp.exp(m_oth[j] - m)
            r = 1.0 / s
            a_own = jnp.exp(m_own - m) * r
            sl = slice(own_base, own_base + v_half)
            out_ref[:, sl] = (out_ref[:, sl] * a_own).astype(out_ref.dtype)
            for j in range(NCH):
                a_j = jnp.exp(m_oth[j] - m) * r
                sl = slice(oth_base + j * CH, oth_base + (j + 1) * CH)
                out_ref[:, sl] = (out_ref[:, sl] * a_j).astype(out_ref.dtype)

        if os.environ.get("KV") == "comm":
            for j in range(NCH):
                rdmas[j].wait_recv()
            out_ref[:, :CH] = comm_ref[0]
        else:
            @pl.when(my_y == 0)
            def _():
                tail(0, v_half)

            @pl.when(my_y == 1)
            def _():
                tail(v_half, 0)

        for k in range(NCH):
            rdmas[k].wait_send()

    return pl.pallas_call(
        body,
        out_shape=jax.ShapeDtypeStruct((t, 2 * v_half), jnp.bfloat16),
        in_specs=[
            pl.BlockSpec(memory_space=pltpu.VMEM),
            pl.BlockSpec(memory_space=pl.ANY),
        ],
        out_specs=pl.BlockSpec(memory_space=pltpu.VMEM),
        scratch_shapes=[
            pltpu.VMEM((NCH, t, CH), jnp.bfloat16),
            pltpu.VMEM((NCH, t, CH), jnp.bfloat16),
            pltpu.VMEM((2, d, CH), jnp.float32),
            pltpu.SemaphoreType.DMA((2,)),
            pltpu.SemaphoreType.DMA((NCH,)),
            pltpu.SemaphoreType.DMA((NCH,)),
        ],
        compiler_params=pltpu.CompilerParams(
            collective_id=0,
            vmem_limit_bytes=67_000_000,
        ),
    )(x, W)


# device time: 115043 ns/iter; 1.4595x vs baseline; 1.0313x over previous
import functools
import os

import jax
import jax.numpy as jnp
from jax import lax
from jax.experimental import pallas as pl
from jax.experimental.pallas import tpu as pltpu


def kernel(x, W):
    t, d = x.shape
    _, v_half = W.shape
    CH = 1024
    NCH = v_half // CH
    f32 = jnp.float32

    def body(x_ref, w_hbm, out_ref, logits_ref, comm_ref, wbuf, wsems,
             send_sems, recv_sems):
        my_x = lax.axis_index("x")
        my_y = lax.axis_index("y")
        my_z = lax.axis_index("z")
        peer = (my_x, 1 - my_y, my_z)

        barrier = pltpu.get_barrier_semaphore()
        pl.semaphore_signal(
            barrier, inc=1, device_id=peer,
            device_id_type=pl.DeviceIdType.MESH,
        )
        pl.semaphore_wait(barrier, 1)

        xb = x_ref[...].astype(jnp.bfloat16)

        def w_copy(k):
            return pltpu.make_async_copy(
                w_hbm.at[:, pl.ds(k * CH, CH)], wbuf.at[k % 2], wsems.at[k % 2]
            )

        rdmas = []
        m_parts = []
        wire_only = os.environ.get("KV") == "wire"
        if not wire_only:
            w_copy(0).start()
        for k in range(NCH):
            if wire_only:
                logits_ref[k] = jnp.broadcast_to(
                    xb[:, :1], (t, CH)).astype(jnp.bfloat16)
                m_parts.append(x_ref[:, :1])
            else:
                if k + 1 < NCH:
                    w_copy(k + 1).start()
                w_copy(k).wait()
                acc = jnp.dot(
                    xb, wbuf[k % 2].astype(jnp.bfloat16),
                    preferred_element_type=f32,
                )
                logits_ref[k] = acc.astype(jnp.bfloat16)
                m_parts.append(acc.max(axis=-1, keepdims=True))
            rdma = pltpu.make_async_remote_copy(
                src_ref=logits_ref.at[k],
                dst_ref=comm_ref.at[k],
                send_sem=send_sems.at[k],
                recv_sem=recv_sems.at[k],
                device_id=peer,
                device_id_type=pl.DeviceIdType.MESH,
            )
            rdma.start()
            rdmas.append(rdma)

        m_own = functools.reduce(jnp.maximum, m_parts)

        def tail(own_base, oth_base):
            s_own = None
            for k in range(NCH):
                e = jnp.exp(logits_ref[k].astype(f32) - m_own)
                out_ref[:, own_base + k * CH:own_base + (k + 1) * CH] = (
                    e.astype(out_ref.dtype))
                es = jnp.sum(e, axis=-1, keepdims=True)
                s_own = es if s_own is None else s_own + es
            m_oth, s_oth = [], []
            for j in range(NCH):
                rdmas[j].wait_recv()
                c = comm_ref[j].astype(f32)
                m_j = c.max(axis=-1, keepdims=True)
                e = jnp.exp(c - m_j)
                out_ref[:, oth_base + j * CH:oth_base + (j + 1) * CH] = (
                    e.astype(out_ref.dtype))
                m_oth.append(m_j)
                s_oth.append(jnp.sum(e, axis=-1, keepdims=True))
            m = functools.reduce(jnp.maximum, m_oth + [m_own])
            s = s_own * jnp.exp(m_own - m)
            for j in range(NCH):
                s = s + s_oth[j] * jnp.exp(m_oth[j] - m)
            r = 1.0 / s
            a_own = jnp.exp(m_own - m) * r
            sl = slice(own_base, own_base + v_half)
            out_ref[:, sl] = (out_ref[:, sl] * a_own).astype(out_ref.dtype)
            for j in range(NCH):
                a_j = jnp.exp(m_oth[j] - m) * r
                sl = slice(oth_base + j * CH, oth_base + (j + 1) * CH)
                out_ref[:, sl] = (out_ref[:, sl] * a_j).astype(out_ref.dtype)

        if os.environ.get("KV") in ("comm", "wire"):
            for j in range(NCH):
                rdmas[j].wait_recv()
            out_ref[:, :CH] = comm_ref[0]
        else:
            @pl.when(my_y == 0)
            def _():
                tail(0, v_half)

            @pl.when(my_y == 1)
            def _():
                tail(v_half, 0)

        for k in range(NCH):
            rdmas[k].wait_send()

    return pl.pallas_call(
        body,
        out_shape=jax.ShapeDtypeStruct((t, 2 * v_half), jnp.bfloat16),
        in_specs=[
            pl.BlockSpec(memory_space=pltpu.VMEM),
            pl.BlockSpec(memory_space=pl.ANY),
        ],
        out_specs=pl.BlockSpec(memory_space=pltpu.VMEM),
        scratch_shapes=[
            pltpu.VMEM((NCH, t, CH), jnp.bfloat16),
            pltpu.VMEM((NCH, t, CH), jnp.bfloat16),
            pltpu.VMEM((2, d, CH), jnp.float32),
            pltpu.SemaphoreType.DMA((2,)),
            pltpu.SemaphoreType.DMA((NCH,)),
            pltpu.SemaphoreType.DMA((NCH,)),
        ],
        compiler_params=pltpu.CompilerParams(
            collective_id=0,
            vmem_limit_bytes=67_000_000,
        ),
    )(x, W)
